# baseline (speedup 1.0000x reference)
import functools

import jax
import jax.numpy as jnp
from jax import lax
from jax.experimental import pallas as pl
from jax.experimental.pallas import tpu as pltpu
from jax.experimental.pallas import tpu_sc as plsc

N = 10000
D = 128
E = 320000

NC = 2
NS = 16
CH = 64
CPT = 160
NT = NC * NS
E_PAD = NT * CPT * CH
N_PAD = 10240
SLAB = N_PAD // NS
RB = 128
NB = N_PAD // RB

_mesh = plsc.VectorSubcoreMesh(core_axis_name="c", subcore_axis_name="s")


@functools.partial(
    pl.kernel,
    out_type=jax.ShapeDtypeStruct((2 * N_PAD, 16), jnp.float32),
    mesh=_mesh,
    scratch_types=[
        pltpu.VMEM((CPT, CH), jnp.int32),
        pltpu.VMEM((CH, 16), jnp.float32),
        pltpu.VMEM((CH, 16), jnp.float32),
        pltpu.VMEM_SHARED((N_PAD, 16), jnp.float32),
        pltpu.SemaphoreType.DMA,
    ],
)
def _sc_degree(dst_hbm, out_hbm, idx_v, ones_v, zeros_v, acc, sem):
    c = lax.axis_index("c")
    s = lax.axis_index("s")
    wid = c * NS + s

    @pl.loop(0, CH)
    def _(i):
        zeros_v[i, :] = jnp.zeros((16,), jnp.float32)

    @pl.loop(0, CH)
    def _(i):
        ones_v[i, :] = jnp.ones((16,), jnp.float32)

    @pl.loop(0, SLAB // CH)
    def _(t):
        pltpu.sync_copy(zeros_v, acc.at[pl.ds(s * SLAB + t * CH, CH)])

    pltpu.async_copy(dst_hbm.at[pl.ds(wid * CPT, CPT)], idx_v, sem).wait()
    plsc.subcore_barrier()

    @pl.loop(0, CPT)
    def _(j):
        pltpu.sync_copy(ones_v, acc.at[idx_v.at[j]], add=True)

    plsc.subcore_barrier()

    @pl.loop(0, SLAB // CH)
    def _(t):
        pltpu.sync_copy(
            acc.at[pl.ds(s * SLAB + t * CH, CH)],
            out_hbm.at[pl.ds(c * N_PAD + s * SLAB + t * CH, CH)],
        )


SEG = 4
CPS = CPT // SEG
NRING = 4


@functools.partial(
    pl.kernel,
    out_type=jax.ShapeDtypeStruct((2 * N_PAD, D), jnp.float32),
    mesh=_mesh,
    scratch_types=[
        pltpu.VMEM((CPS, CH), jnp.int32),
        pltpu.VMEM((CPS, CH), jnp.int32),
    ] + [pltpu.VMEM((CH, D), jnp.float32)] * NRING
      + [pltpu.VMEM_SHARED((N_PAD, D), jnp.float32)]
      + [pltpu.SemaphoreType.DMA] * (NRING + 1),
)
def _sc_edge(g_hbm, src_hbm, dst_hbm, out_hbm, src_v, dst_v, *rest):
    rows = rest[:NRING]
    acc = rest[NRING]
    sem_i = rest[NRING + 1]
    sems = rest[NRING + 2:]
    c = lax.axis_index("c")
    s = lax.axis_index("s")
    wid = c * NS + s

    @pl.loop(0, CH)
    def _(i):
        @pl.loop(0, D // 16)
        def _(k):
            rows[0][i, pl.ds(k * 16, 16)] = jnp.zeros((16,), jnp.float32)

    @pl.loop(0, SLAB // CH)
    def _(t):
        pltpu.sync_copy(rows[0], acc.at[pl.ds(s * SLAB + t * CH, CH)])

    plsc.subcore_barrier()

    @pl.loop(0, SEG)
    def _(g):
        base = wid * CPT + g * CPS
        pltpu.async_copy(src_hbm.at[pl.ds(base, CPS)], src_v, sem_i).wait()
        pltpu.async_copy(dst_hbm.at[pl.ds(base, CPS)], dst_v, sem_i).wait()

        for b in range(NRING):
            pltpu.async_copy(g_hbm.at[src_v.at[b]], rows[b], sems[b])

        @pl.loop(0, CPS, step=NRING)
        def _(j):
            for b in range(NRING):
                pltpu.make_async_copy(
                    g_hbm.at[src_v.at[j + b]], rows[b], sems[b]).wait()
                pltpu.sync_copy(rows[b], acc.at[dst_v.at[j + b]], add=True)

                @pl.when(j + b + NRING < CPS)
                def _():
                    pltpu.async_copy(
                        g_hbm.at[src_v.at[j + b + NRING]], rows[b], sems[b])

    plsc.subcore_barrier()

    @pl.loop(0, SLAB // CH)
    def _(t):
        pltpu.sync_copy(
            acc.at[pl.ds(s * SLAB + t * CH, CH)],
            out_hbm.at[pl.ds(c * N_PAD + s * SLAB + t * CH, CH)],
        )


def _prep_body(x_ref, w_ref, d0_ref, d1_ref, dis_ref, g_ref):
    deg = d0_ref[:, 0:1] + d1_ref[:, 0:1] + 1.0
    dis = lax.rsqrt(deg)
    dis_ref[...] = dis
    g_ref[...] = dis * jnp.dot(
        x_ref[...], w_ref[...], preferred_element_type=jnp.float32)


def _tc_prep(x_pad, W1, dpart):
    return pl.pallas_call(
        _prep_body,
        grid=(NB,),
        in_specs=[
            pl.BlockSpec((RB, D), lambda i: (i, 0)),
            pl.BlockSpec((D, D), lambda i: (0, 0)),
            pl.BlockSpec((RB, 16), lambda i: (i, 0)),
            pl.BlockSpec((RB, 16), lambda i: (NB + i, 0)),
        ],
        out_specs=[
            pl.BlockSpec((RB, 1), lambda i: (i, 0)),
            pl.BlockSpec((RB, D), lambda i: (i, 0)),
        ],
        out_shape=[
            jax.ShapeDtypeStruct((N_PAD, 1), jnp.float32),
            jax.ShapeDtypeStruct((N_PAD, D), jnp.float32),
        ],
    )(x_pad, W1, dpart, dpart)


def _mid_body(p0_ref, p1_ref, g_ref, dis_ref, b_ref, w_ref, o_ref):
    dis = dis_ref[...]
    z = jnp.maximum(
        dis * (p0_ref[...] + p1_ref[...] + g_ref[...]) + b_ref[...], 0.0)
    o_ref[...] = dis * jnp.dot(
        z, w_ref[...], preferred_element_type=jnp.float32)


def _tc_mid(part, g, dis, b, W2):
    return pl.pallas_call(
        _mid_body,
        grid=(NB,),
        in_specs=[
            pl.BlockSpec((RB, D), lambda i: (i, 0)),
            pl.BlockSpec((RB, D), lambda i: (NB + i, 0)),
            pl.BlockSpec((RB, D), lambda i: (i, 0)),
            pl.BlockSpec((RB, 1), lambda i: (i, 0)),
            pl.BlockSpec((1, D), lambda i: (0, 0)),
            pl.BlockSpec((D, D), lambda i: (0, 0)),
        ],
        out_specs=pl.BlockSpec((RB, D), lambda i: (i, 0)),
        out_shape=jax.ShapeDtypeStruct((N_PAD, D), jnp.float32),
    )(part, part, g, dis, b, W2)


def _out_body(p0_ref, p1_ref, g_ref, dis_ref, b_ref, o_ref):
    o_ref[...] = jnp.maximum(
        dis_ref[...] * (p0_ref[...] + p1_ref[...] + g_ref[...]) + b_ref[...],
        0.0)


def _tc_out(part, g, dis, b):
    return pl.pallas_call(
        _out_body,
        grid=(NB,),
        in_specs=[
            pl.BlockSpec((RB, D), lambda i: (i, 0)),
            pl.BlockSpec((RB, D), lambda i: (NB + i, 0)),
            pl.BlockSpec((RB, D), lambda i: (i, 0)),
            pl.BlockSpec((RB, 1), lambda i: (i, 0)),
            pl.BlockSpec((1, D), lambda i: (0, 0)),
        ],
        out_specs=pl.BlockSpec((RB, D), lambda i: (i, 0)),
        out_shape=jax.ShapeDtypeStruct((N_PAD, D), jnp.float32),
    )(part, part, g, dis, b)


def kernel(x, edge_index, W1, b1, W2, b2):
    src = edge_index[0].astype(jnp.int32)
    dst = edge_index[1].astype(jnp.int32)
    pad = E_PAD - E
    src_p = jnp.concatenate([src, jnp.zeros((pad,), jnp.int32)])
    dst_p = jnp.concatenate(
        [dst, N + (jnp.arange(pad, dtype=jnp.int32) % (N_PAD - N))])
    src2d = src_p.reshape(NT * CPT, CH)
    dst2d = dst_p.reshape(NT * CPT, CH)

    x_pad = jnp.zeros((N_PAD, D), jnp.float32).at[:N].set(x)
    b1r = b1.reshape(1, D)
    b2r = b2.reshape(1, D)

    dpart = _sc_degree(dst2d)
    dis, g1 = _tc_prep(x_pad, W1, dpart)
    part1 = _sc_edge(g1, src2d, dst2d)
    g2 = _tc_mid(part1, g1, dis, b1r, W2)
    part2 = _sc_edge(g2, src2d, dst2d)
    h2 = _tc_out(part2, g2, dis, b2r)
    return h2[:N]


# speedup vs baseline: 1.0599x; 1.0599x over previous
import functools

import jax
import jax.numpy as jnp
from jax import lax
from jax.experimental import pallas as pl
from jax.experimental.pallas import tpu as pltpu
from jax.experimental.pallas import tpu_sc as plsc

N = 10000
D = 128
E = 320000

NC = 2
NS = 16
CH = 128
CPT = 80
NT = NC * NS
E_PAD = NT * CPT * CH
N_PAD = 10240
SLAB = N_PAD // NS
RB = 128
NB = N_PAD // RB

_mesh = plsc.VectorSubcoreMesh(core_axis_name="c", subcore_axis_name="s")


@functools.partial(
    pl.kernel,
    out_type=jax.ShapeDtypeStruct((2 * N_PAD, 16), jnp.float32),
    mesh=_mesh,
    scratch_types=[
        pltpu.VMEM((CPT, CH), jnp.int32),
        pltpu.VMEM((CH, 16), jnp.float32),
        pltpu.VMEM((CH, 16), jnp.float32),
        pltpu.VMEM_SHARED((N_PAD, 16), jnp.float32),
        pltpu.SemaphoreType.DMA,
    ],
)
def _sc_degree(dst_hbm, out_hbm, idx_v, ones_v, zeros_v, acc, sem):
    c = lax.axis_index("c")
    s = lax.axis_index("s")
    wid = c * NS + s

    @pl.loop(0, CH)
    def _(i):
        zeros_v[i, :] = jnp.zeros((16,), jnp.float32)

    @pl.loop(0, CH)
    def _(i):
        ones_v[i, :] = jnp.ones((16,), jnp.float32)

    @pl.loop(0, SLAB // CH)
    def _(t):
        pltpu.sync_copy(zeros_v, acc.at[pl.ds(s * SLAB + t * CH, CH)])

    pltpu.async_copy(dst_hbm.at[pl.ds(wid * CPT, CPT)], idx_v, sem).wait()
    plsc.subcore_barrier()

    @pl.loop(0, CPT)
    def _(j):
        pltpu.sync_copy(ones_v, acc.at[idx_v.at[j]], add=True)

    plsc.subcore_barrier()

    @pl.loop(0, SLAB // CH)
    def _(t):
        pltpu.sync_copy(
            acc.at[pl.ds(s * SLAB + t * CH, CH)],
            out_hbm.at[pl.ds(c * N_PAD + s * SLAB + t * CH, CH)],
        )


SEG = 2
CPS = CPT // SEG
NSPLIT = 2
QW = CH // NSPLIT


def _gather_chunk(g_hbm, src_v, buf, jj, sems):
    for q in range(NSPLIT):
        pltpu.async_copy(
            g_hbm.at[src_v.at[jj, pl.ds(q * QW, QW)]],
            buf.at[pl.ds(q * QW, QW)], sems[q])


def _wait_chunk(g_hbm, src_v, buf, jj, sems):
    for q in range(NSPLIT):
        pltpu.make_async_copy(
            g_hbm.at[src_v.at[jj, pl.ds(q * QW, QW)]],
            buf.at[pl.ds(q * QW, QW)], sems[q]).wait()


@functools.partial(
    pl.kernel,
    out_type=jax.ShapeDtypeStruct((2 * N_PAD, D), jnp.float32),
    mesh=_mesh,
    scratch_types=[
        pltpu.VMEM((CPS, CH), jnp.int32),
        pltpu.VMEM((CPS, CH), jnp.int32),
        pltpu.VMEM((CH, D), jnp.float32),
        pltpu.VMEM((CH, D), jnp.float32),
        pltpu.VMEM_SHARED((N_PAD, D), jnp.float32),
    ] + [pltpu.SemaphoreType.DMA] * (1 + 2 * NSPLIT),
)
def _sc_edge(g_hbm, src_hbm, dst_hbm, out_hbm,
             src_v, dst_v, rows0, rows1, acc, sem_i, *gsems):
    sems = (gsems[:NSPLIT], gsems[NSPLIT:])
    rows = (rows0, rows1)
    c = lax.axis_index("c")
    s = lax.axis_index("s")
    wid = c * NS + s

    @pl.loop(0, CH)
    def _(i):
        @pl.loop(0, D // 16)
        def _(k):
            rows0[i, pl.ds(k * 16, 16)] = jnp.zeros((16,), jnp.float32)

    @pl.loop(0, SLAB // CH)
    def _(t):
        pltpu.sync_copy(rows0, acc.at[pl.ds(s * SLAB + t * CH, CH)])

    plsc.subcore_barrier()

    @pl.loop(0, SEG)
    def _(g):
        base = wid * CPT + g * CPS
        pltpu.async_copy(src_hbm.at[pl.ds(base, CPS)], src_v, sem_i).wait()
        pltpu.async_copy(dst_hbm.at[pl.ds(base, CPS)], dst_v, sem_i).wait()

        _gather_chunk(g_hbm, src_v, rows0, 0, sems[0])
        _gather_chunk(g_hbm, src_v, rows1, 1, sems[1])

        @pl.loop(0, CPS, step=2)
        def _(j):
            for b in range(2):
                jj = j + b
                _wait_chunk(g_hbm, src_v, rows[b], jj, sems[b])
                pltpu.sync_copy(rows[b], acc.at[dst_v.at[jj]], add=True)

                @pl.when(jj + 2 < CPS)
                def _():
                    _gather_chunk(g_hbm, src_v, rows[b], jj + 2, sems[b])

    plsc.subcore_barrier()

    @pl.loop(0, SLAB // CH)
    def _(t):
        pltpu.sync_copy(
            acc.at[pl.ds(s * SLAB + t * CH, CH)],
            out_hbm.at[pl.ds(c * N_PAD + s * SLAB + t * CH, CH)],
        )


def _prep_body(x_ref, w_ref, d0_ref, d1_ref, dis_ref, g_ref):
    deg = d0_ref[:, 0:1] + d1_ref[:, 0:1] + 1.0
    dis = lax.rsqrt(deg)
    dis_ref[...] = dis
    g_ref[...] = dis * jnp.dot(
        x_ref[...], w_ref[...], preferred_element_type=jnp.float32)


def _tc_prep(x_pad, W1, dpart):
    return pl.pallas_call(
        _prep_body,
        grid=(NB,),
        in_specs=[
            pl.BlockSpec((RB, D), lambda i: (i, 0)),
            pl.BlockSpec((D, D), lambda i: (0, 0)),
            pl.BlockSpec((RB, 16), lambda i: (i, 0)),
            pl.BlockSpec((RB, 16), lambda i: (NB + i, 0)),
        ],
        out_specs=[
            pl.BlockSpec((RB, 1), lambda i: (i, 0)),
            pl.BlockSpec((RB, D), lambda i: (i, 0)),
        ],
        out_shape=[
            jax.ShapeDtypeStruct((N_PAD, 1), jnp.float32),
            jax.ShapeDtypeStruct((N_PAD, D), jnp.float32),
        ],
    )(x_pad, W1, dpart, dpart)


def _mid_body(p0_ref, p1_ref, g_ref, dis_ref, b_ref, w_ref, o_ref):
    dis = dis_ref[...]
    z = jnp.maximum(
        dis * (p0_ref[...] + p1_ref[...] + g_ref[...]) + b_ref[...], 0.0)
    o_ref[...] = dis * jnp.dot(
        z, w_ref[...], preferred_element_type=jnp.float32)


def _tc_mid(part, g, dis, b, W2):
    return pl.pallas_call(
        _mid_body,
        grid=(NB,),
        in_specs=[
            pl.BlockSpec((RB, D), lambda i: (i, 0)),
            pl.BlockSpec((RB, D), lambda i: (NB + i, 0)),
            pl.BlockSpec((RB, D), lambda i: (i, 0)),
            pl.BlockSpec((RB, 1), lambda i: (i, 0)),
            pl.BlockSpec((1, D), lambda i: (0, 0)),
            pl.BlockSpec((D, D), lambda i: (0, 0)),
        ],
        out_specs=pl.BlockSpec((RB, D), lambda i: (i, 0)),
        out_shape=jax.ShapeDtypeStruct((N_PAD, D), jnp.float32),
    )(part, part, g, dis, b, W2)


def _out_body(p0_ref, p1_ref, g_ref, dis_ref, b_ref, o_ref):
    o_ref[...] = jnp.maximum(
        dis_ref[...] * (p0_ref[...] + p1_ref[...] + g_ref[...]) + b_ref[...],
        0.0)


def _tc_out(part, g, dis, b):
    return pl.pallas_call(
        _out_body,
        grid=(NB,),
        in_specs=[
            pl.BlockSpec((RB, D), lambda i: (i, 0)),
            pl.BlockSpec((RB, D), lambda i: (NB + i, 0)),
            pl.BlockSpec((RB, D), lambda i: (i, 0)),
            pl.BlockSpec((RB, 1), lambda i: (i, 0)),
            pl.BlockSpec((1, D), lambda i: (0, 0)),
        ],
        out_specs=pl.BlockSpec((RB, D), lambda i: (i, 0)),
        out_shape=jax.ShapeDtypeStruct((N_PAD, D), jnp.float32),
    )(part, part, g, dis, b)


def kernel(x, edge_index, W1, b1, W2, b2):
    src = edge_index[0].astype(jnp.int32)
    dst = edge_index[1].astype(jnp.int32)
    pad = E_PAD - E
    src_p = jnp.concatenate([src, jnp.zeros((pad,), jnp.int32)])
    dst_p = jnp.concatenate(
        [dst, N + (jnp.arange(pad, dtype=jnp.int32) % (N_PAD - N))])
    src2d = src_p.reshape(NT * CPT, CH)
    dst2d = dst_p.reshape(NT * CPT, CH)

    x_pad = jnp.zeros((N_PAD, D), jnp.float32).at[:N].set(x)
    b1r = b1.reshape(1, D)
    b2r = b2.reshape(1, D)

    dpart = _sc_degree(dst2d)
    dis, g1 = _tc_prep(x_pad, W1, dpart)
    part1 = _sc_edge(g1, src2d, dst2d)
    g2 = _tc_mid(part1, g1, dis, b1r, W2)
    part2 = _sc_edge(g2, src2d, dst2d)
    h2 = _tc_out(part2, g2, dis, b2r)
    return h2[:N]
